# baseline (speedup 1.0000x reference)
import functools

import jax
import jax.numpy as jnp
from jax import lax
from jax.experimental import pallas as pl
from jax.experimental.pallas import tpu as pltpu, tpu_sc as plsc

N_SITES = 50000
N_IN = 128
N_OUT = 128
FV = 9
FVA = 4
FVB = FV - FVA

NW = 32
CHUNK = 1568
SUB = 784
CHUNK_L = N_SITES - (NW - 1) * CHUNK
SUB_L = CHUNK_L // 2
BN = 1024


def _mm_body(nf, feat_ref, w_ref, b_ref, out_ref):
    f = feat_ref[...]
    for k in range(nf):
        out_ref[k] = (
            jnp.dot(f, w_ref[k], preferred_element_type=jnp.float32)
            + b_ref[0] * (1.0 / FV)
        )


def _transform(features, weight, bias):
    nf = weight.shape[0]
    grid = (pl.cdiv(N_SITES, BN),)
    return pl.pallas_call(
        functools.partial(_mm_body, nf),
        grid=grid,
        in_specs=[
            pl.BlockSpec((BN, N_IN), lambda i: (i, 0)),
            pl.BlockSpec((nf, N_IN, N_OUT), lambda i: (0, 0, 0)),
            pl.BlockSpec((1, N_OUT), lambda i: (0, 0)),
        ],
        out_specs=pl.BlockSpec((nf, BN, N_OUT), lambda i: (0, i, 0)),
        out_shape=jax.ShapeDtypeStruct((nf, N_SITES, N_OUT), jnp.float32),
    )(features, weight, bias.reshape(1, N_OUT))


def _work(nf, pa_hbm, t_hbm, idx_hbm, out_hbm, idx_v, acc_v, sem,
          base, chunk, sub):
    for f in range(nf):
        pltpu.sync_copy(
            idx_hbm.at[pl.ds(f * N_SITES + base, chunk)],
            idx_v.at[pl.ds(f * chunk, chunk)],
        )
    for i in range(chunk // sub):
        off = base + i * sub
        acc = acc_v.at[pl.ds(0, sub)]
        if pa_hbm is None:
            first = 1
            pltpu.async_copy(
                t_hbm.at[idx_v.at[pl.ds(i * sub, sub)]], acc, sem
            ).wait()
        else:
            first = 0
            pltpu.sync_copy(pa_hbm.at[pl.ds(off, sub)], acc)
        for f in range(first, nf):
            pltpu.async_copy(
                t_hbm.at[idx_v.at[pl.ds(f * chunk + i * sub, sub)]],
                acc,
                sem,
                add=True,
            ).wait()
        pltpu.sync_copy(acc, out_hbm.at[pl.ds(off, sub)])


def _sc_body(nf, with_pa, *refs):
    if with_pa:
        t_hbm, idx_hbm, pa_hbm, out_hbm, idx_v, acc_v, sem = refs
    else:
        t_hbm, idx_hbm, out_hbm, idx_v, acc_v, sem = refs
        pa_hbm = None
    c = lax.axis_index("c")
    s = lax.axis_index("s")
    wid = s * 2 + c
    base = wid * CHUNK

    @pl.when(wid < NW - 1)
    def _full():
        _work(nf, pa_hbm, t_hbm, idx_hbm, out_hbm, idx_v, acc_v, sem,
              base, CHUNK, SUB)

    @pl.when(wid == NW - 1)
    def _last():
        _work(nf, pa_hbm, t_hbm, idx_hbm, out_hbm, idx_v, acc_v, sem,
              base, CHUNK_L, SUB_L)


def _make_gather(nf, with_pa):
    return functools.partial(
        pl.kernel,
        out_type=jax.ShapeDtypeStruct((N_SITES, N_OUT), jnp.float32),
        mesh=plsc.VectorSubcoreMesh(core_axis_name="c", subcore_axis_name="s"),
        scratch_types=[
            pltpu.VMEM((nf * CHUNK,), jnp.int32),
            pltpu.VMEM((SUB, N_OUT), jnp.float32),
            pltpu.SemaphoreType.DMA,
        ],
    )(functools.partial(_sc_body, nf, with_pa))


_gather_a = _make_gather(FVA, False)
_gather_b = _make_gather(FVB, True)


@jax.jit
def kernel(features, neighbor_idx, weight, bias):
    idx_t = (
        neighbor_idx.T
        + (jnp.arange(FV, dtype=jnp.int32) * N_SITES)[:, None]
    )
    idx_a = idx_t[:FVA].reshape(FVA * N_SITES)
    idx_b = (idx_t[FVA:] - FVA * N_SITES).reshape(FVB * N_SITES)

    t_a = _transform(features, weight[:FVA], bias)
    t_b = _transform(features, weight[FVA:], bias)
    pa = _gather_a(t_a.reshape(FVA * N_SITES, N_OUT), idx_a)
    return _gather_b(t_b.reshape(FVB * N_SITES, N_OUT), idx_b, pa)
